# baseline (speedup 1.0000x reference)
import functools

import jax
import jax.numpy as jnp
from jax import lax
from jax.experimental import pallas as pl
from jax.experimental.pallas import tpu as pltpu
from jax.experimental.pallas import tpu_sc as plsc

_CHUNK = 128


@functools.lru_cache(maxsize=None)
def _make_sc_gather(total, vocab, dim):
    info = plsc.get_sparse_core_info()
    num_cores = info.num_cores
    num_workers = info.num_cores * info.num_subcores
    b_per_w = total // num_workers
    n_ch = b_per_w // _CHUNK
    mesh = plsc.VectorSubcoreMesh(core_axis_name="c", subcore_axis_name="s")

    @functools.partial(
        pl.kernel,
        mesh=mesh,
        out_type=jax.ShapeDtypeStruct((num_workers, n_ch, _CHUNK, dim),
                                      jnp.float32),
        scratch_types=[
            pltpu.VMEM((n_ch, _CHUNK), jnp.int32),
            pltpu.VMEM((n_ch, _CHUNK, dim), jnp.float32),
            pltpu.SemaphoreType.DMA,
        ],
        compiler_params=pltpu.CompilerParams(use_tc_tiling_on_sc=False),
    )
    def sc_gather(ids_hbm, table_hbm, out_hbm, idx_v, rows_v, gsem):
        wid = lax.axis_index("s") * num_cores + lax.axis_index("c")
        pltpu.sync_copy(ids_hbm.at[wid], idx_v)
        handles = [
            pltpu.async_copy(table_hbm.at[idx_v.at[j]], rows_v.at[j], gsem)
            for j in range(n_ch)
        ]
        for h in handles:
            h.wait()
        pltpu.sync_copy(rows_v, out_hbm.at[wid])

    return sc_gather


def kernel(input_ids, embedding_table):
    batch, seq = input_ids.shape
    vocab, dim = embedding_table.shape
    total = batch * seq
    info = plsc.get_sparse_core_info()
    num_workers = info.num_cores * info.num_subcores
    ids3 = input_ids.astype(jnp.int32).reshape(
        num_workers, total // num_workers // _CHUNK, _CHUNK)
    gathered = _make_sc_gather(total, vocab, dim)(ids3, embedding_table)
    output = gathered.reshape(batch, seq, dim)
    return (output, embedding_table)
